# baseline (speedup 1.0000x reference)
import jax
import jax.numpy as jnp
from jax import lax
from jax.experimental import pallas as pl
from jax.experimental.pallas import tpu as pltpu
from jax.experimental.pallas import tpu_sc as plsc

EMB = 64
SEQ = 200
NW = 32
CHUNK = 128
N_CHUNKS = 200
SCALE = 8.0
NLANE = 16


def _body(ids_hbm, tok_hbm, pos_hbm, out_hbm, idx_v, pos_v, gbuf, sem):
    cid = lax.axis_index("c")
    sid = lax.axis_index("s")
    wid = cid * 16 + sid
    pltpu.sync_copy(ids_hbm.at[wid], idx_v)
    pltpu.sync_copy(pos_hbm, pos_v)
    row0 = wid * (N_CHUNKS * CHUNK)

    def chunk_body(c, carry):
        pltpu.async_copy(tok_hbm.at[idx_v.at[c]], gbuf, sem).wait()
        p0 = lax.rem(c * CHUNK, SEQ)

        def row_body(j, p):
            for k in range(EMB // NLANE):
                g = gbuf[j, pl.ds(NLANE * k, NLANE)]
                pv = pos_v[p, pl.ds(NLANE * k, NLANE)]
                gbuf[j, pl.ds(NLANE * k, NLANE)] = SCALE * g + pv
            p = p + 1
            return jnp.where(p == SEQ, 0, p)

        lax.fori_loop(0, CHUNK, row_body, p0)
        pltpu.sync_copy(gbuf, out_hbm.at[pl.ds(row0 + c * CHUNK, CHUNK)])
        return carry

    lax.fori_loop(0, N_CHUNKS, chunk_body, 0)


def kernel(input_ids, tok_table, pos_table):
    batch, seq = input_ids.shape
    ids = input_ids.reshape(NW, N_CHUNKS, CHUNK).astype(jnp.int32)
    mesh = plsc.VectorSubcoreMesh(core_axis_name="c", subcore_axis_name="s")
    out = pl.kernel(
        _body,
        out_type=jax.ShapeDtypeStruct((NW * N_CHUNKS * CHUNK, EMB), jnp.float32),
        mesh=mesh,
        compiler_params=pltpu.CompilerParams(use_tc_tiling_on_sc=False),
        scratch_types=[
            pltpu.VMEM((N_CHUNKS, CHUNK), jnp.int32),
            pltpu.VMEM((SEQ, EMB), jnp.float32),
            pltpu.VMEM((CHUNK, EMB), jnp.float32),
            pltpu.SemaphoreType.DMA,
        ],
    )(ids, tok_table, pos_table)
    return out.reshape(batch, seq, EMB)
